# baseline (speedup 1.0000x reference)
import functools

import jax
import jax.numpy as jnp
from jax import lax
from jax.experimental import pallas as pl
from jax.experimental.pallas import tpu as pltpu
from jax.experimental.pallas import tpu_sc as plsc

_NC = 2
_NS = 16
_CHUNK = 128
_D = 128
_SPLIT0 = 0.25


def _sc_aggregate(x_hbm, src, dst, zeros_hbm):
    n_pad = x_hbm.shape[0]
    e_pad = src.shape[0] - _CHUNK
    tot = e_pad // (_NS * _CHUNK)
    cpt0 = int(tot * _SPLIT0 + 0.5)
    cpt1 = tot - cpt0
    rpt = n_pad // _NS

    mesh = plsc.VectorSubcoreMesh(core_axis_name="c", subcore_axis_name="s")

    @functools.partial(
        pl.kernel,
        out_type=jax.ShapeDtypeStruct((_NC, n_pad, _D), jnp.float32),
        mesh=mesh,
        scratch_types=[
            pltpu.VMEM((_CHUNK,), jnp.int32),
            pltpu.VMEM((_CHUNK,), jnp.int32),
            pltpu.VMEM((_CHUNK, _D), jnp.float32),
            pltpu.VMEM_SHARED((n_pad, _D), jnp.float32),
            pltpu.SemaphoreType.DMA,
            pltpu.SemaphoreType.DMA,
        ],
    )
    def agg_kernel(x_h, src_h, dst_h, z_h, out_h, sidx, didx, rows, acc, sem,
                   ssem):
        cid = lax.axis_index("c")
        sid = lax.axis_index("s")
        r0 = sid * rpt
        cpt = jnp.where(cid == 0, cpt0, cpt1)
        ebase = jnp.where(cid == 0, sid * cpt0,
                          _NS * cpt0 + sid * cpt1) * _CHUNK
        dummy = pl.multiple_of(e_pad, _CHUNK)
        pltpu.sync_copy(z_h.at[pl.ds(r0, rpt)], acc.at[pl.ds(r0, rpt)])
        plsc.subcore_barrier()
        pltpu.sync_copy(dst_h.at[pl.ds(dummy, _CHUNK)], didx)
        pltpu.async_copy(rows, acc.at[didx], ssem, add=True)

        def body(c, carry):
            off = pl.multiple_of(ebase + c * _CHUNK, _CHUNK)
            pltpu.sync_copy(src_h.at[pl.ds(off, _CHUNK)], sidx)
            pltpu.make_async_copy(rows, acc.at[didx], ssem).wait()
            g = pltpu.async_copy(x_h.at[sidx], rows, sem)
            pltpu.sync_copy(dst_h.at[pl.ds(off, _CHUNK)], didx)
            g.wait()
            pltpu.async_copy(rows, acc.at[didx], ssem, add=True)
            return carry

        lax.fori_loop(0, cpt, body, 0)
        pltpu.make_async_copy(rows, acc.at[didx], ssem).wait()
        plsc.subcore_barrier()
        pltpu.sync_copy(acc.at[pl.ds(r0, rpt)], out_h.at[cid, pl.ds(r0, rpt)])

    return agg_kernel(x_hbm, src, dst, zeros_hbm)


def _mlp(parts, x, eps, W1, b1, W2, b2):
    n_pad = x.shape[0]
    blk = 1024
    eps_arr = jnp.reshape(eps, (1, 1)).astype(jnp.float32)

    def body(eps_ref, p_ref, x_ref, w1_ref, b1_ref, w2_ref, b2_ref, o_ref):
        a = p_ref[0] + p_ref[1] + (1.0 + eps_ref[0, 0]) * x_ref[...]
        h = jnp.dot(a, w1_ref[...], preferred_element_type=jnp.float32) + b1_ref[...]
        h = jnp.maximum(h, 0.0)
        o_ref[...] = jnp.dot(h, w2_ref[...], preferred_element_type=jnp.float32) + b2_ref[...]

    return pl.pallas_call(
        body,
        grid=(n_pad // blk,),
        in_specs=[
            pl.BlockSpec(memory_space=pltpu.SMEM),
            pl.BlockSpec((_NC, blk, _D), lambda i: (0, i, 0)),
            pl.BlockSpec((blk, _D), lambda i: (i, 0)),
            pl.BlockSpec((_D, _D), lambda i: (0, 0)),
            pl.BlockSpec((1, _D), lambda i: (0, 0)),
            pl.BlockSpec((_D, _D), lambda i: (0, 0)),
            pl.BlockSpec((1, _D), lambda i: (0, 0)),
        ],
        out_specs=pl.BlockSpec((blk, _D), lambda i: (i, 0)),
        out_shape=jax.ShapeDtypeStruct((n_pad, _D), jnp.float32),
    )(eps_arr, parts, x, W1, b1.reshape(1, _D), W2, b2.reshape(1, _D))


def _mlp_final(parts, x2, eps, W1, b1, W2, b2, x0, h1, Wf, bf):
    n_pad = x2.shape[0]
    blk = 1024
    eps_arr = jnp.reshape(eps, (1, 1)).astype(jnp.float32)

    def body(eps_ref, p_ref, x2_ref, w1_ref, b1_ref, w2_ref, b2_ref,
             x0_ref, h1_ref, wf_ref, bf_ref, o_ref):
        a = p_ref[0] + p_ref[1] + (1.0 + eps_ref[0, 0]) * x2_ref[...]
        t = jnp.dot(a, w1_ref[...], preferred_element_type=jnp.float32) + b1_ref[...]
        t = jnp.maximum(t, 0.0)
        h3 = jnp.dot(t, w2_ref[...], preferred_element_type=jnp.float32) + b2_ref[...]
        acc = jnp.dot(x0_ref[...], wf_ref[0:_D], preferred_element_type=jnp.float32)
        acc += jnp.dot(h1_ref[...], wf_ref[_D:2 * _D], preferred_element_type=jnp.float32)
        acc += jnp.dot(x2_ref[...], wf_ref[2 * _D:3 * _D], preferred_element_type=jnp.float32)
        acc += jnp.dot(h3, wf_ref[3 * _D:4 * _D], preferred_element_type=jnp.float32)
        o_ref[...] = acc + bf_ref[...]

    return pl.pallas_call(
        body,
        grid=(n_pad // blk,),
        in_specs=[
            pl.BlockSpec(memory_space=pltpu.SMEM),
            pl.BlockSpec((_NC, blk, _D), lambda i: (0, i, 0)),
            pl.BlockSpec((blk, _D), lambda i: (i, 0)),
            pl.BlockSpec((_D, _D), lambda i: (0, 0)),
            pl.BlockSpec((1, _D), lambda i: (0, 0)),
            pl.BlockSpec((_D, _D), lambda i: (0, 0)),
            pl.BlockSpec((1, _D), lambda i: (0, 0)),
            pl.BlockSpec((blk, _D), lambda i: (i, 0)),
            pl.BlockSpec((blk, _D), lambda i: (i, 0)),
            pl.BlockSpec((4 * _D, _D), lambda i: (0, 0)),
            pl.BlockSpec((1, _D), lambda i: (0, 0)),
        ],
        out_specs=pl.BlockSpec((blk, _D), lambda i: (i, 0)),
        out_shape=jax.ShapeDtypeStruct((n_pad, _D), jnp.float32),
    )(eps_arr, parts, x2, W1, b1.reshape(1, _D), W2, b2.reshape(1, _D),
      x0, h1, Wf, bf.reshape(1, _D))


def kernel(x, edge_index, eps0, W1_0, b1_0, W2_0, b2_0, eps1, W1_1, b1_1,
           W2_1, b2_1, eps2, W1_2, b1_2, W2_2, b2_2, Wf, bf):
    n = x.shape[0]
    e = edge_index.shape[1]
    blk = 1024
    n_pad = -(-(n + 1) // blk) * blk
    egrain = _NS * _CHUNK
    e_pad = -(-e // egrain) * egrain

    src = edge_index[0].astype(jnp.int32)
    dst = edge_index[1].astype(jnp.int32)
    pad_idx = jnp.full((e_pad - e + _CHUNK,), n, dtype=jnp.int32)
    src_p = jnp.concatenate([src, pad_idx])
    dst_p = jnp.concatenate([dst, pad_idx])

    x_pad = jnp.concatenate([x, jnp.zeros((n_pad - n, _D), jnp.float32)])
    zeros_hbm = jnp.zeros((n_pad, _D), jnp.float32)

    parts1 = _sc_aggregate(x_pad, src_p, dst_p, zeros_hbm)
    h1 = _mlp(parts1, x_pad, eps0, W1_0, b1_0, W2_0, b2_0)
    parts2 = _sc_aggregate(h1, src_p, dst_p, zeros_hbm)
    h2 = _mlp(parts2, h1, eps1, W1_1, b1_1, W2_1, b2_1)
    parts3 = _sc_aggregate(h2, src_p, dst_p, zeros_hbm)
    out_pad = _mlp_final(parts3, h2, eps2, W1_2, b1_2, W2_2, b2_2,
                         x_pad, h1, Wf, bf)
    return out_pad[:n]


# speedup vs baseline: 1.0665x; 1.0665x over previous
import functools

import jax
import jax.numpy as jnp
from jax import lax
from jax.experimental import pallas as pl
from jax.experimental.pallas import tpu as pltpu
from jax.experimental.pallas import tpu_sc as plsc

_NC = 2
_NS = 16
_CHUNK = 128
_D = 128
_SPLIT0 = 0.32


def _sc_aggregate(x_hbm, src, dst, zeros_hbm):
    n_pad = x_hbm.shape[0]
    e_pad = src.shape[0] - _CHUNK
    tot = e_pad // (_NS * _CHUNK)
    cpt0 = int(tot * _SPLIT0 + 0.5)
    cpt1 = tot - cpt0
    rpt = n_pad // _NS

    mesh = plsc.VectorSubcoreMesh(core_axis_name="c", subcore_axis_name="s")

    @functools.partial(
        pl.kernel,
        out_type=jax.ShapeDtypeStruct((_NC, n_pad, _D), jnp.float32),
        mesh=mesh,
        scratch_types=[
            pltpu.VMEM((_CHUNK,), jnp.int32),
            pltpu.VMEM((_CHUNK,), jnp.int32),
            pltpu.VMEM((_CHUNK, _D), jnp.float32),
            pltpu.VMEM_SHARED((n_pad, _D), jnp.float32),
            pltpu.SemaphoreType.DMA,
            pltpu.SemaphoreType.DMA,
        ],
    )
    def agg_kernel(x_h, src_h, dst_h, z_h, out_h, sidx, didx, rows, acc, sem,
                   ssem):
        cid = lax.axis_index("c")
        sid = lax.axis_index("s")
        r0 = sid * rpt
        cpt = jnp.where(cid == 0, cpt0, cpt1)
        ebase = jnp.where(cid == 0, sid * cpt0,
                          _NS * cpt0 + sid * cpt1) * _CHUNK
        dummy = pl.multiple_of(e_pad, _CHUNK)
        pltpu.sync_copy(z_h.at[pl.ds(r0, rpt)], acc.at[pl.ds(r0, rpt)])
        plsc.subcore_barrier()
        pltpu.sync_copy(dst_h.at[pl.ds(dummy, _CHUNK)], didx)
        pltpu.async_copy(rows, acc.at[didx], ssem, add=True)

        def body(c, carry):
            off = pl.multiple_of(ebase + c * _CHUNK, _CHUNK)
            pltpu.sync_copy(src_h.at[pl.ds(off, _CHUNK)], sidx)
            pltpu.make_async_copy(rows, acc.at[didx], ssem).wait()
            g = pltpu.async_copy(x_h.at[sidx], rows, sem)
            pltpu.sync_copy(dst_h.at[pl.ds(off, _CHUNK)], didx)
            g.wait()
            pltpu.async_copy(rows, acc.at[didx], ssem, add=True)
            return carry

        lax.fori_loop(0, cpt, body, 0)
        pltpu.make_async_copy(rows, acc.at[didx], ssem).wait()
        plsc.subcore_barrier()
        pltpu.sync_copy(acc.at[pl.ds(r0, rpt)], out_h.at[cid, pl.ds(r0, rpt)])

    return agg_kernel(x_hbm, src, dst, zeros_hbm)


def _mlp(parts, x, eps, W1, b1, W2, b2):
    n_pad = x.shape[0]
    blk = 1024
    eps_arr = jnp.reshape(eps, (1, 1)).astype(jnp.float32)

    def body(eps_ref, p_ref, x_ref, w1_ref, b1_ref, w2_ref, b2_ref, o_ref):
        a = p_ref[0] + p_ref[1] + (1.0 + eps_ref[0, 0]) * x_ref[...]
        h = jnp.dot(a, w1_ref[...], preferred_element_type=jnp.float32) + b1_ref[...]
        h = jnp.maximum(h, 0.0)
        o_ref[...] = jnp.dot(h, w2_ref[...], preferred_element_type=jnp.float32) + b2_ref[...]

    return pl.pallas_call(
        body,
        grid=(n_pad // blk,),
        in_specs=[
            pl.BlockSpec(memory_space=pltpu.SMEM),
            pl.BlockSpec((_NC, blk, _D), lambda i: (0, i, 0)),
            pl.BlockSpec((blk, _D), lambda i: (i, 0)),
            pl.BlockSpec((_D, _D), lambda i: (0, 0)),
            pl.BlockSpec((1, _D), lambda i: (0, 0)),
            pl.BlockSpec((_D, _D), lambda i: (0, 0)),
            pl.BlockSpec((1, _D), lambda i: (0, 0)),
        ],
        out_specs=pl.BlockSpec((blk, _D), lambda i: (i, 0)),
        out_shape=jax.ShapeDtypeStruct((n_pad, _D), jnp.float32),
    )(eps_arr, parts, x, W1, b1.reshape(1, _D), W2, b2.reshape(1, _D))


def _mlp_final(parts, x2, eps, W1, b1, W2, b2, x0, h1, Wf, bf):
    n_pad = x2.shape[0]
    blk = 1024
    eps_arr = jnp.reshape(eps, (1, 1)).astype(jnp.float32)

    def body(eps_ref, p_ref, x2_ref, w1_ref, b1_ref, w2_ref, b2_ref,
             x0_ref, h1_ref, wf_ref, bf_ref, o_ref):
        a = p_ref[0] + p_ref[1] + (1.0 + eps_ref[0, 0]) * x2_ref[...]
        t = jnp.dot(a, w1_ref[...], preferred_element_type=jnp.float32) + b1_ref[...]
        t = jnp.maximum(t, 0.0)
        h3 = jnp.dot(t, w2_ref[...], preferred_element_type=jnp.float32) + b2_ref[...]
        acc = jnp.dot(x0_ref[...], wf_ref[0:_D], preferred_element_type=jnp.float32)
        acc += jnp.dot(h1_ref[...], wf_ref[_D:2 * _D], preferred_element_type=jnp.float32)
        acc += jnp.dot(x2_ref[...], wf_ref[2 * _D:3 * _D], preferred_element_type=jnp.float32)
        acc += jnp.dot(h3, wf_ref[3 * _D:4 * _D], preferred_element_type=jnp.float32)
        o_ref[...] = acc + bf_ref[...]

    return pl.pallas_call(
        body,
        grid=(n_pad // blk,),
        in_specs=[
            pl.BlockSpec(memory_space=pltpu.SMEM),
            pl.BlockSpec((_NC, blk, _D), lambda i: (0, i, 0)),
            pl.BlockSpec((blk, _D), lambda i: (i, 0)),
            pl.BlockSpec((_D, _D), lambda i: (0, 0)),
            pl.BlockSpec((1, _D), lambda i: (0, 0)),
            pl.BlockSpec((_D, _D), lambda i: (0, 0)),
            pl.BlockSpec((1, _D), lambda i: (0, 0)),
            pl.BlockSpec((blk, _D), lambda i: (i, 0)),
            pl.BlockSpec((blk, _D), lambda i: (i, 0)),
            pl.BlockSpec((4 * _D, _D), lambda i: (0, 0)),
            pl.BlockSpec((1, _D), lambda i: (0, 0)),
        ],
        out_specs=pl.BlockSpec((blk, _D), lambda i: (i, 0)),
        out_shape=jax.ShapeDtypeStruct((n_pad, _D), jnp.float32),
    )(eps_arr, parts, x2, W1, b1.reshape(1, _D), W2, b2.reshape(1, _D),
      x0, h1, Wf, bf.reshape(1, _D))


def kernel(x, edge_index, eps0, W1_0, b1_0, W2_0, b2_0, eps1, W1_1, b1_1,
           W2_1, b2_1, eps2, W1_2, b1_2, W2_2, b2_2, Wf, bf):
    n = x.shape[0]
    e = edge_index.shape[1]
    blk = 1024
    n_pad = -(-(n + 1) // blk) * blk
    egrain = _NS * _CHUNK
    e_pad = -(-e // egrain) * egrain

    src = edge_index[0].astype(jnp.int32)
    dst = edge_index[1].astype(jnp.int32)
    pad_idx = jnp.full((e_pad - e + _CHUNK,), n, dtype=jnp.int32)
    src_p = jnp.concatenate([src, pad_idx])
    dst_p = jnp.concatenate([dst, pad_idx])

    x_pad = jnp.concatenate([x, jnp.zeros((n_pad - n, _D), jnp.float32)])
    zeros_hbm = jnp.zeros((n_pad, _D), jnp.float32)

    parts1 = _sc_aggregate(x_pad, src_p, dst_p, zeros_hbm)
    h1 = _mlp(parts1, x_pad, eps0, W1_0, b1_0, W2_0, b2_0)
    parts2 = _sc_aggregate(h1, src_p, dst_p, zeros_hbm)
    h2 = _mlp(parts2, h1, eps1, W1_1, b1_1, W2_1, b2_1)
    parts3 = _sc_aggregate(h2, src_p, dst_p, zeros_hbm)
    out_pad = _mlp_final(parts3, h2, eps2, W1_2, b1_2, W2_2, b2_2,
                         x_pad, h1, Wf, bf)
    return out_pad[:n]


# speedup vs baseline: 1.1496x; 1.0779x over previous
import functools

import jax
import jax.numpy as jnp
from jax import lax
from jax.experimental import pallas as pl
from jax.experimental.pallas import tpu as pltpu
from jax.experimental.pallas import tpu_sc as plsc

_NC = 2
_NS = 16
_CHUNK = 128
_D = 128
_SPLIT0 = 0.40


def _sc_aggregate(x_hbm, src, dst, zeros_hbm):
    n_pad = x_hbm.shape[0]
    e_pad = src.shape[0] - _CHUNK
    tot = e_pad // (_NS * _CHUNK)
    cpt0 = int(tot * _SPLIT0 + 0.5)
    cpt1 = tot - cpt0
    rpt = n_pad // _NS

    mesh = plsc.VectorSubcoreMesh(core_axis_name="c", subcore_axis_name="s")

    @functools.partial(
        pl.kernel,
        out_type=jax.ShapeDtypeStruct((_NC, n_pad, _D), jnp.float32),
        mesh=mesh,
        scratch_types=[
            pltpu.VMEM((_CHUNK,), jnp.int32),
            pltpu.VMEM((_CHUNK,), jnp.int32),
            pltpu.VMEM((_CHUNK, _D), jnp.float32),
            pltpu.VMEM_SHARED((n_pad, _D), jnp.float32),
            pltpu.SemaphoreType.DMA,
            pltpu.SemaphoreType.DMA,
        ],
    )
    def agg_kernel(x_h, src_h, dst_h, z_h, out_h, sidx, didx, rows, acc, sem,
                   ssem):
        cid = lax.axis_index("c")
        sid = lax.axis_index("s")
        r0 = sid * rpt
        cpt = jnp.where(cid == 0, cpt0, cpt1)
        ebase = jnp.where(cid == 0, sid * cpt0,
                          _NS * cpt0 + sid * cpt1) * _CHUNK
        dummy = pl.multiple_of(e_pad, _CHUNK)
        pltpu.sync_copy(z_h.at[pl.ds(r0, rpt)], acc.at[pl.ds(r0, rpt)])
        plsc.subcore_barrier()
        pltpu.sync_copy(dst_h.at[pl.ds(dummy, _CHUNK)], didx)
        pltpu.async_copy(rows, acc.at[didx], ssem, add=True)

        def body(c, carry):
            off = pl.multiple_of(ebase + c * _CHUNK, _CHUNK)
            pltpu.sync_copy(src_h.at[pl.ds(off, _CHUNK)], sidx)
            pltpu.make_async_copy(rows, acc.at[didx], ssem).wait()
            g = pltpu.async_copy(x_h.at[sidx], rows, sem)
            pltpu.sync_copy(dst_h.at[pl.ds(off, _CHUNK)], didx)
            g.wait()
            pltpu.async_copy(rows, acc.at[didx], ssem, add=True)
            return carry

        lax.fori_loop(0, cpt, body, 0)
        pltpu.make_async_copy(rows, acc.at[didx], ssem).wait()
        plsc.subcore_barrier()
        pltpu.sync_copy(acc.at[pl.ds(r0, rpt)], out_h.at[cid, pl.ds(r0, rpt)])

    return agg_kernel(x_hbm, src, dst, zeros_hbm)


def _mlp(parts, x, eps, W1, b1, W2, b2):
    n_pad = x.shape[0]
    blk = 1024
    eps_arr = jnp.reshape(eps, (1, 1)).astype(jnp.float32)

    def body(eps_ref, p_ref, x_ref, w1_ref, b1_ref, w2_ref, b2_ref, o_ref):
        a = p_ref[0] + p_ref[1] + (1.0 + eps_ref[0, 0]) * x_ref[...]
        h = jnp.dot(a, w1_ref[...], preferred_element_type=jnp.float32) + b1_ref[...]
        h = jnp.maximum(h, 0.0)
        o_ref[...] = jnp.dot(h, w2_ref[...], preferred_element_type=jnp.float32) + b2_ref[...]

    return pl.pallas_call(
        body,
        grid=(n_pad // blk,),
        in_specs=[
            pl.BlockSpec(memory_space=pltpu.SMEM),
            pl.BlockSpec((_NC, blk, _D), lambda i: (0, i, 0)),
            pl.BlockSpec((blk, _D), lambda i: (i, 0)),
            pl.BlockSpec((_D, _D), lambda i: (0, 0)),
            pl.BlockSpec((1, _D), lambda i: (0, 0)),
            pl.BlockSpec((_D, _D), lambda i: (0, 0)),
            pl.BlockSpec((1, _D), lambda i: (0, 0)),
        ],
        out_specs=pl.BlockSpec((blk, _D), lambda i: (i, 0)),
        out_shape=jax.ShapeDtypeStruct((n_pad, _D), jnp.float32),
    )(eps_arr, parts, x, W1, b1.reshape(1, _D), W2, b2.reshape(1, _D))


def _mlp_final(parts, x2, eps, W1, b1, W2, b2, x0, h1, Wf, bf):
    n_pad = x2.shape[0]
    blk = 1024
    eps_arr = jnp.reshape(eps, (1, 1)).astype(jnp.float32)

    def body(eps_ref, p_ref, x2_ref, w1_ref, b1_ref, w2_ref, b2_ref,
             x0_ref, h1_ref, wf_ref, bf_ref, o_ref):
        a = p_ref[0] + p_ref[1] + (1.0 + eps_ref[0, 0]) * x2_ref[...]
        t = jnp.dot(a, w1_ref[...], preferred_element_type=jnp.float32) + b1_ref[...]
        t = jnp.maximum(t, 0.0)
        h3 = jnp.dot(t, w2_ref[...], preferred_element_type=jnp.float32) + b2_ref[...]
        acc = jnp.dot(x0_ref[...], wf_ref[0:_D], preferred_element_type=jnp.float32)
        acc += jnp.dot(h1_ref[...], wf_ref[_D:2 * _D], preferred_element_type=jnp.float32)
        acc += jnp.dot(x2_ref[...], wf_ref[2 * _D:3 * _D], preferred_element_type=jnp.float32)
        acc += jnp.dot(h3, wf_ref[3 * _D:4 * _D], preferred_element_type=jnp.float32)
        o_ref[...] = acc + bf_ref[...]

    return pl.pallas_call(
        body,
        grid=(n_pad // blk,),
        in_specs=[
            pl.BlockSpec(memory_space=pltpu.SMEM),
            pl.BlockSpec((_NC, blk, _D), lambda i: (0, i, 0)),
            pl.BlockSpec((blk, _D), lambda i: (i, 0)),
            pl.BlockSpec((_D, _D), lambda i: (0, 0)),
            pl.BlockSpec((1, _D), lambda i: (0, 0)),
            pl.BlockSpec((_D, _D), lambda i: (0, 0)),
            pl.BlockSpec((1, _D), lambda i: (0, 0)),
            pl.BlockSpec((blk, _D), lambda i: (i, 0)),
            pl.BlockSpec((blk, _D), lambda i: (i, 0)),
            pl.BlockSpec((4 * _D, _D), lambda i: (0, 0)),
            pl.BlockSpec((1, _D), lambda i: (0, 0)),
        ],
        out_specs=pl.BlockSpec((blk, _D), lambda i: (i, 0)),
        out_shape=jax.ShapeDtypeStruct((n_pad, _D), jnp.float32),
    )(eps_arr, parts, x2, W1, b1.reshape(1, _D), W2, b2.reshape(1, _D),
      x0, h1, Wf, bf.reshape(1, _D))


def kernel(x, edge_index, eps0, W1_0, b1_0, W2_0, b2_0, eps1, W1_1, b1_1,
           W2_1, b2_1, eps2, W1_2, b1_2, W2_2, b2_2, Wf, bf):
    n = x.shape[0]
    e = edge_index.shape[1]
    blk = 1024
    n_pad = -(-(n + 1) // blk) * blk
    egrain = _NS * _CHUNK
    e_pad = -(-e // egrain) * egrain

    src = edge_index[0].astype(jnp.int32)
    dst = edge_index[1].astype(jnp.int32)
    pad_idx = jnp.full((e_pad - e + _CHUNK,), n, dtype=jnp.int32)
    src_p = jnp.concatenate([src, pad_idx])
    dst_p = jnp.concatenate([dst, pad_idx])

    x_pad = jnp.concatenate([x, jnp.zeros((n_pad - n, _D), jnp.float32)])
    zeros_hbm = jnp.zeros((n_pad, _D), jnp.float32)

    parts1 = _sc_aggregate(x_pad, src_p, dst_p, zeros_hbm)
    h1 = _mlp(parts1, x_pad, eps0, W1_0, b1_0, W2_0, b2_0)
    parts2 = _sc_aggregate(h1, src_p, dst_p, zeros_hbm)
    h2 = _mlp(parts2, h1, eps1, W1_1, b1_1, W2_1, b2_1)
    parts3 = _sc_aggregate(h2, src_p, dst_p, zeros_hbm)
    out_pad = _mlp_final(parts3, h2, eps2, W1_2, b1_2, W2_2, b2_2,
                         x_pad, h1, Wf, bf)
    return out_pad[:n]


# speedup vs baseline: 1.1944x; 1.0390x over previous
import functools

import jax
import jax.numpy as jnp
from jax import lax
from jax.experimental import pallas as pl
from jax.experimental.pallas import tpu as pltpu
from jax.experimental.pallas import tpu_sc as plsc

_NC = 2
_NS = 16
_CHUNK = 128
_D = 128
_SPLIT0 = 0.44


def _sc_aggregate(x_hbm, src, dst, zeros_hbm):
    n_pad = x_hbm.shape[0]
    e_pad = src.shape[0] - _CHUNK
    tot = e_pad // (_NS * _CHUNK)
    cpt0 = int(tot * _SPLIT0 + 0.5)
    cpt1 = tot - cpt0
    rpt = n_pad // _NS

    mesh = plsc.VectorSubcoreMesh(core_axis_name="c", subcore_axis_name="s")

    @functools.partial(
        pl.kernel,
        out_type=jax.ShapeDtypeStruct((_NC, n_pad, _D), jnp.float32),
        mesh=mesh,
        scratch_types=[
            pltpu.VMEM((_CHUNK,), jnp.int32),
            pltpu.VMEM((_CHUNK,), jnp.int32),
            pltpu.VMEM((_CHUNK, _D), jnp.float32),
            pltpu.VMEM_SHARED((n_pad, _D), jnp.float32),
            pltpu.SemaphoreType.DMA,
            pltpu.SemaphoreType.DMA,
        ],
    )
    def agg_kernel(x_h, src_h, dst_h, z_h, out_h, sidx, didx, rows, acc, sem,
                   ssem):
        cid = lax.axis_index("c")
        sid = lax.axis_index("s")
        r0 = sid * rpt
        cpt = jnp.where(cid == 0, cpt0, cpt1)
        ebase = jnp.where(cid == 0, sid * cpt0,
                          _NS * cpt0 + sid * cpt1) * _CHUNK
        dummy = pl.multiple_of(e_pad, _CHUNK)
        pltpu.sync_copy(z_h.at[pl.ds(r0, rpt)], acc.at[pl.ds(r0, rpt)])
        plsc.subcore_barrier()
        pltpu.sync_copy(dst_h.at[pl.ds(dummy, _CHUNK)], didx)
        pltpu.async_copy(rows, acc.at[didx], ssem, add=True)

        def body(c, carry):
            off = pl.multiple_of(ebase + c * _CHUNK, _CHUNK)
            pltpu.sync_copy(src_h.at[pl.ds(off, _CHUNK)], sidx)
            pltpu.make_async_copy(rows, acc.at[didx], ssem).wait()
            g = pltpu.async_copy(x_h.at[sidx], rows, sem)
            pltpu.sync_copy(dst_h.at[pl.ds(off, _CHUNK)], didx)
            g.wait()
            pltpu.async_copy(rows, acc.at[didx], ssem, add=True)
            return carry

        lax.fori_loop(0, cpt, body, 0)
        pltpu.make_async_copy(rows, acc.at[didx], ssem).wait()
        plsc.subcore_barrier()
        pltpu.sync_copy(acc.at[pl.ds(r0, rpt)], out_h.at[cid, pl.ds(r0, rpt)])

    return agg_kernel(x_hbm, src, dst, zeros_hbm)


def _mlp(parts, x, eps, W1, b1, W2, b2):
    n_pad = x.shape[0]
    blk = 1024
    eps_arr = jnp.reshape(eps, (1, 1)).astype(jnp.float32)

    def body(eps_ref, p_ref, x_ref, w1_ref, b1_ref, w2_ref, b2_ref, o_ref):
        a = p_ref[0] + p_ref[1] + (1.0 + eps_ref[0, 0]) * x_ref[...]
        h = jnp.dot(a, w1_ref[...], preferred_element_type=jnp.float32) + b1_ref[...]
        h = jnp.maximum(h, 0.0)
        o_ref[...] = jnp.dot(h, w2_ref[...], preferred_element_type=jnp.float32) + b2_ref[...]

    return pl.pallas_call(
        body,
        grid=(n_pad // blk,),
        in_specs=[
            pl.BlockSpec(memory_space=pltpu.SMEM),
            pl.BlockSpec((_NC, blk, _D), lambda i: (0, i, 0)),
            pl.BlockSpec((blk, _D), lambda i: (i, 0)),
            pl.BlockSpec((_D, _D), lambda i: (0, 0)),
            pl.BlockSpec((1, _D), lambda i: (0, 0)),
            pl.BlockSpec((_D, _D), lambda i: (0, 0)),
            pl.BlockSpec((1, _D), lambda i: (0, 0)),
        ],
        out_specs=pl.BlockSpec((blk, _D), lambda i: (i, 0)),
        out_shape=jax.ShapeDtypeStruct((n_pad, _D), jnp.float32),
    )(eps_arr, parts, x, W1, b1.reshape(1, _D), W2, b2.reshape(1, _D))


def _mlp_final(parts, x2, eps, W1, b1, W2, b2, x0, h1, Wf, bf):
    n_pad = x2.shape[0]
    blk = 1024
    eps_arr = jnp.reshape(eps, (1, 1)).astype(jnp.float32)

    def body(eps_ref, p_ref, x2_ref, w1_ref, b1_ref, w2_ref, b2_ref,
             x0_ref, h1_ref, wf_ref, bf_ref, o_ref):
        a = p_ref[0] + p_ref[1] + (1.0 + eps_ref[0, 0]) * x2_ref[...]
        t = jnp.dot(a, w1_ref[...], preferred_element_type=jnp.float32) + b1_ref[...]
        t = jnp.maximum(t, 0.0)
        h3 = jnp.dot(t, w2_ref[...], preferred_element_type=jnp.float32) + b2_ref[...]
        acc = jnp.dot(x0_ref[...], wf_ref[0:_D], preferred_element_type=jnp.float32)
        acc += jnp.dot(h1_ref[...], wf_ref[_D:2 * _D], preferred_element_type=jnp.float32)
        acc += jnp.dot(x2_ref[...], wf_ref[2 * _D:3 * _D], preferred_element_type=jnp.float32)
        acc += jnp.dot(h3, wf_ref[3 * _D:4 * _D], preferred_element_type=jnp.float32)
        o_ref[...] = acc + bf_ref[...]

    return pl.pallas_call(
        body,
        grid=(n_pad // blk,),
        in_specs=[
            pl.BlockSpec(memory_space=pltpu.SMEM),
            pl.BlockSpec((_NC, blk, _D), lambda i: (0, i, 0)),
            pl.BlockSpec((blk, _D), lambda i: (i, 0)),
            pl.BlockSpec((_D, _D), lambda i: (0, 0)),
            pl.BlockSpec((1, _D), lambda i: (0, 0)),
            pl.BlockSpec((_D, _D), lambda i: (0, 0)),
            pl.BlockSpec((1, _D), lambda i: (0, 0)),
            pl.BlockSpec((blk, _D), lambda i: (i, 0)),
            pl.BlockSpec((blk, _D), lambda i: (i, 0)),
            pl.BlockSpec((4 * _D, _D), lambda i: (0, 0)),
            pl.BlockSpec((1, _D), lambda i: (0, 0)),
        ],
        out_specs=pl.BlockSpec((blk, _D), lambda i: (i, 0)),
        out_shape=jax.ShapeDtypeStruct((n_pad, _D), jnp.float32),
    )(eps_arr, parts, x2, W1, b1.reshape(1, _D), W2, b2.reshape(1, _D),
      x0, h1, Wf, bf.reshape(1, _D))


def kernel(x, edge_index, eps0, W1_0, b1_0, W2_0, b2_0, eps1, W1_1, b1_1,
           W2_1, b2_1, eps2, W1_2, b1_2, W2_2, b2_2, Wf, bf):
    n = x.shape[0]
    e = edge_index.shape[1]
    blk = 1024
    n_pad = -(-(n + 1) // blk) * blk
    egrain = _NS * _CHUNK
    e_pad = -(-e // egrain) * egrain

    src = edge_index[0].astype(jnp.int32)
    dst = edge_index[1].astype(jnp.int32)
    pad_idx = jnp.full((e_pad - e + _CHUNK,), n, dtype=jnp.int32)
    src_p = jnp.concatenate([src, pad_idx])
    dst_p = jnp.concatenate([dst, pad_idx])

    x_pad = jnp.concatenate([x, jnp.zeros((n_pad - n, _D), jnp.float32)])
    zeros_hbm = jnp.zeros((n_pad, _D), jnp.float32)

    parts1 = _sc_aggregate(x_pad, src_p, dst_p, zeros_hbm)
    h1 = _mlp(parts1, x_pad, eps0, W1_0, b1_0, W2_0, b2_0)
    parts2 = _sc_aggregate(h1, src_p, dst_p, zeros_hbm)
    h2 = _mlp(parts2, h1, eps1, W1_1, b1_1, W2_1, b2_1)
    parts3 = _sc_aggregate(h2, src_p, dst_p, zeros_hbm)
    out_pad = _mlp_final(parts3, h2, eps2, W1_2, b1_2, W2_2, b2_2,
                         x_pad, h1, Wf, bf)
    return out_pad[:n]


# speedup vs baseline: 1.2350x; 1.0340x over previous
import functools

import jax
import jax.numpy as jnp
from jax import lax
from jax.experimental import pallas as pl
from jax.experimental.pallas import tpu as pltpu
from jax.experimental.pallas import tpu_sc as plsc

_NC = 2
_NS = 16
_CHUNK = 128
_D = 128
_SPLIT0 = 0.48


def _sc_aggregate(x_hbm, src, dst, zeros_hbm):
    n_pad = x_hbm.shape[0]
    e_pad = src.shape[0] - _CHUNK
    tot = e_pad // (_NS * _CHUNK)
    cpt0 = int(tot * _SPLIT0 + 0.5)
    cpt1 = tot - cpt0
    rpt = n_pad // _NS

    mesh = plsc.VectorSubcoreMesh(core_axis_name="c", subcore_axis_name="s")

    @functools.partial(
        pl.kernel,
        out_type=jax.ShapeDtypeStruct((_NC, n_pad, _D), jnp.float32),
        mesh=mesh,
        scratch_types=[
            pltpu.VMEM((_CHUNK,), jnp.int32),
            pltpu.VMEM((_CHUNK,), jnp.int32),
            pltpu.VMEM((_CHUNK, _D), jnp.float32),
            pltpu.VMEM_SHARED((n_pad, _D), jnp.float32),
            pltpu.SemaphoreType.DMA,
            pltpu.SemaphoreType.DMA,
        ],
    )
    def agg_kernel(x_h, src_h, dst_h, z_h, out_h, sidx, didx, rows, acc, sem,
                   ssem):
        cid = lax.axis_index("c")
        sid = lax.axis_index("s")
        r0 = sid * rpt
        cpt = jnp.where(cid == 0, cpt0, cpt1)
        ebase = jnp.where(cid == 0, sid * cpt0,
                          _NS * cpt0 + sid * cpt1) * _CHUNK
        dummy = pl.multiple_of(e_pad, _CHUNK)
        pltpu.sync_copy(z_h.at[pl.ds(r0, rpt)], acc.at[pl.ds(r0, rpt)])
        plsc.subcore_barrier()
        pltpu.sync_copy(dst_h.at[pl.ds(dummy, _CHUNK)], didx)
        pltpu.async_copy(rows, acc.at[didx], ssem, add=True)

        def body(c, carry):
            off = pl.multiple_of(ebase + c * _CHUNK, _CHUNK)
            pltpu.sync_copy(src_h.at[pl.ds(off, _CHUNK)], sidx)
            pltpu.make_async_copy(rows, acc.at[didx], ssem).wait()
            g = pltpu.async_copy(x_h.at[sidx], rows, sem)
            pltpu.sync_copy(dst_h.at[pl.ds(off, _CHUNK)], didx)
            g.wait()
            pltpu.async_copy(rows, acc.at[didx], ssem, add=True)
            return carry

        lax.fori_loop(0, cpt, body, 0)
        pltpu.make_async_copy(rows, acc.at[didx], ssem).wait()
        plsc.subcore_barrier()
        pltpu.sync_copy(acc.at[pl.ds(r0, rpt)], out_h.at[cid, pl.ds(r0, rpt)])

    return agg_kernel(x_hbm, src, dst, zeros_hbm)


def _mlp(parts, x, eps, W1, b1, W2, b2):
    n_pad = x.shape[0]
    blk = 1024
    eps_arr = jnp.reshape(eps, (1, 1)).astype(jnp.float32)

    def body(eps_ref, p_ref, x_ref, w1_ref, b1_ref, w2_ref, b2_ref, o_ref):
        a = p_ref[0] + p_ref[1] + (1.0 + eps_ref[0, 0]) * x_ref[...]
        h = jnp.dot(a, w1_ref[...], preferred_element_type=jnp.float32) + b1_ref[...]
        h = jnp.maximum(h, 0.0)
        o_ref[...] = jnp.dot(h, w2_ref[...], preferred_element_type=jnp.float32) + b2_ref[...]

    return pl.pallas_call(
        body,
        grid=(n_pad // blk,),
        in_specs=[
            pl.BlockSpec(memory_space=pltpu.SMEM),
            pl.BlockSpec((_NC, blk, _D), lambda i: (0, i, 0)),
            pl.BlockSpec((blk, _D), lambda i: (i, 0)),
            pl.BlockSpec((_D, _D), lambda i: (0, 0)),
            pl.BlockSpec((1, _D), lambda i: (0, 0)),
            pl.BlockSpec((_D, _D), lambda i: (0, 0)),
            pl.BlockSpec((1, _D), lambda i: (0, 0)),
        ],
        out_specs=pl.BlockSpec((blk, _D), lambda i: (i, 0)),
        out_shape=jax.ShapeDtypeStruct((n_pad, _D), jnp.float32),
    )(eps_arr, parts, x, W1, b1.reshape(1, _D), W2, b2.reshape(1, _D))


def _mlp_final(parts, x2, eps, W1, b1, W2, b2, x0, h1, Wf, bf):
    n_pad = x2.shape[0]
    blk = 1024
    eps_arr = jnp.reshape(eps, (1, 1)).astype(jnp.float32)

    def body(eps_ref, p_ref, x2_ref, w1_ref, b1_ref, w2_ref, b2_ref,
             x0_ref, h1_ref, wf_ref, bf_ref, o_ref):
        a = p_ref[0] + p_ref[1] + (1.0 + eps_ref[0, 0]) * x2_ref[...]
        t = jnp.dot(a, w1_ref[...], preferred_element_type=jnp.float32) + b1_ref[...]
        t = jnp.maximum(t, 0.0)
        h3 = jnp.dot(t, w2_ref[...], preferred_element_type=jnp.float32) + b2_ref[...]
        acc = jnp.dot(x0_ref[...], wf_ref[0:_D], preferred_element_type=jnp.float32)
        acc += jnp.dot(h1_ref[...], wf_ref[_D:2 * _D], preferred_element_type=jnp.float32)
        acc += jnp.dot(x2_ref[...], wf_ref[2 * _D:3 * _D], preferred_element_type=jnp.float32)
        acc += jnp.dot(h3, wf_ref[3 * _D:4 * _D], preferred_element_type=jnp.float32)
        o_ref[...] = acc + bf_ref[...]

    return pl.pallas_call(
        body,
        grid=(n_pad // blk,),
        in_specs=[
            pl.BlockSpec(memory_space=pltpu.SMEM),
            pl.BlockSpec((_NC, blk, _D), lambda i: (0, i, 0)),
            pl.BlockSpec((blk, _D), lambda i: (i, 0)),
            pl.BlockSpec((_D, _D), lambda i: (0, 0)),
            pl.BlockSpec((1, _D), lambda i: (0, 0)),
            pl.BlockSpec((_D, _D), lambda i: (0, 0)),
            pl.BlockSpec((1, _D), lambda i: (0, 0)),
            pl.BlockSpec((blk, _D), lambda i: (i, 0)),
            pl.BlockSpec((blk, _D), lambda i: (i, 0)),
            pl.BlockSpec((4 * _D, _D), lambda i: (0, 0)),
            pl.BlockSpec((1, _D), lambda i: (0, 0)),
        ],
        out_specs=pl.BlockSpec((blk, _D), lambda i: (i, 0)),
        out_shape=jax.ShapeDtypeStruct((n_pad, _D), jnp.float32),
    )(eps_arr, parts, x2, W1, b1.reshape(1, _D), W2, b2.reshape(1, _D),
      x0, h1, Wf, bf.reshape(1, _D))


def kernel(x, edge_index, eps0, W1_0, b1_0, W2_0, b2_0, eps1, W1_1, b1_1,
           W2_1, b2_1, eps2, W1_2, b1_2, W2_2, b2_2, Wf, bf):
    n = x.shape[0]
    e = edge_index.shape[1]
    blk = 1024
    n_pad = -(-(n + 1) // blk) * blk
    egrain = _NS * _CHUNK
    e_pad = -(-e // egrain) * egrain

    src = edge_index[0].astype(jnp.int32)
    dst = edge_index[1].astype(jnp.int32)
    pad_idx = jnp.full((e_pad - e + _CHUNK,), n, dtype=jnp.int32)
    src_p = jnp.concatenate([src, pad_idx])
    dst_p = jnp.concatenate([dst, pad_idx])

    x_pad = jnp.concatenate([x, jnp.zeros((n_pad - n, _D), jnp.float32)])
    zeros_hbm = jnp.zeros((n_pad, _D), jnp.float32)

    parts1 = _sc_aggregate(x_pad, src_p, dst_p, zeros_hbm)
    h1 = _mlp(parts1, x_pad, eps0, W1_0, b1_0, W2_0, b2_0)
    parts2 = _sc_aggregate(h1, src_p, dst_p, zeros_hbm)
    h2 = _mlp(parts2, h1, eps1, W1_1, b1_1, W2_1, b2_1)
    parts3 = _sc_aggregate(h2, src_p, dst_p, zeros_hbm)
    out_pad = _mlp_final(parts3, h2, eps2, W1_2, b1_2, W2_2, b2_2,
                         x_pad, h1, Wf, bf)
    return out_pad[:n]


# speedup vs baseline: 1.2768x; 1.0339x over previous
import functools

import jax
import jax.numpy as jnp
from jax import lax
from jax.experimental import pallas as pl
from jax.experimental.pallas import tpu as pltpu
from jax.experimental.pallas import tpu_sc as plsc

_NC = 2
_NS = 16
_CHUNK = 128
_D = 128
_SPLIT0 = 0.52


def _sc_aggregate(x_hbm, src, dst, zeros_hbm):
    n_pad = x_hbm.shape[0]
    e_pad = src.shape[0] - _CHUNK
    tot = e_pad // (_NS * _CHUNK)
    cpt0 = int(tot * _SPLIT0 + 0.5)
    cpt1 = tot - cpt0
    rpt = n_pad // _NS

    mesh = plsc.VectorSubcoreMesh(core_axis_name="c", subcore_axis_name="s")

    @functools.partial(
        pl.kernel,
        out_type=jax.ShapeDtypeStruct((_NC, n_pad, _D), jnp.float32),
        mesh=mesh,
        scratch_types=[
            pltpu.VMEM((_CHUNK,), jnp.int32),
            pltpu.VMEM((_CHUNK,), jnp.int32),
            pltpu.VMEM((_CHUNK, _D), jnp.float32),
            pltpu.VMEM_SHARED((n_pad, _D), jnp.float32),
            pltpu.SemaphoreType.DMA,
            pltpu.SemaphoreType.DMA,
        ],
    )
    def agg_kernel(x_h, src_h, dst_h, z_h, out_h, sidx, didx, rows, acc, sem,
                   ssem):
        cid = lax.axis_index("c")
        sid = lax.axis_index("s")
        r0 = sid * rpt
        cpt = jnp.where(cid == 0, cpt0, cpt1)
        ebase = jnp.where(cid == 0, sid * cpt0,
                          _NS * cpt0 + sid * cpt1) * _CHUNK
        dummy = pl.multiple_of(e_pad, _CHUNK)
        pltpu.sync_copy(z_h.at[pl.ds(r0, rpt)], acc.at[pl.ds(r0, rpt)])
        plsc.subcore_barrier()
        pltpu.sync_copy(dst_h.at[pl.ds(dummy, _CHUNK)], didx)
        pltpu.async_copy(rows, acc.at[didx], ssem, add=True)

        def body(c, carry):
            off = pl.multiple_of(ebase + c * _CHUNK, _CHUNK)
            pltpu.sync_copy(src_h.at[pl.ds(off, _CHUNK)], sidx)
            pltpu.make_async_copy(rows, acc.at[didx], ssem).wait()
            g = pltpu.async_copy(x_h.at[sidx], rows, sem)
            pltpu.sync_copy(dst_h.at[pl.ds(off, _CHUNK)], didx)
            g.wait()
            pltpu.async_copy(rows, acc.at[didx], ssem, add=True)
            return carry

        lax.fori_loop(0, cpt, body, 0)
        pltpu.make_async_copy(rows, acc.at[didx], ssem).wait()
        plsc.subcore_barrier()
        pltpu.sync_copy(acc.at[pl.ds(r0, rpt)], out_h.at[cid, pl.ds(r0, rpt)])

    return agg_kernel(x_hbm, src, dst, zeros_hbm)


def _mlp(parts, x, eps, W1, b1, W2, b2):
    n_pad = x.shape[0]
    blk = 1024
    eps_arr = jnp.reshape(eps, (1, 1)).astype(jnp.float32)

    def body(eps_ref, p_ref, x_ref, w1_ref, b1_ref, w2_ref, b2_ref, o_ref):
        a = p_ref[0] + p_ref[1] + (1.0 + eps_ref[0, 0]) * x_ref[...]
        h = jnp.dot(a, w1_ref[...], preferred_element_type=jnp.float32) + b1_ref[...]
        h = jnp.maximum(h, 0.0)
        o_ref[...] = jnp.dot(h, w2_ref[...], preferred_element_type=jnp.float32) + b2_ref[...]

    return pl.pallas_call(
        body,
        grid=(n_pad // blk,),
        in_specs=[
            pl.BlockSpec(memory_space=pltpu.SMEM),
            pl.BlockSpec((_NC, blk, _D), lambda i: (0, i, 0)),
            pl.BlockSpec((blk, _D), lambda i: (i, 0)),
            pl.BlockSpec((_D, _D), lambda i: (0, 0)),
            pl.BlockSpec((1, _D), lambda i: (0, 0)),
            pl.BlockSpec((_D, _D), lambda i: (0, 0)),
            pl.BlockSpec((1, _D), lambda i: (0, 0)),
        ],
        out_specs=pl.BlockSpec((blk, _D), lambda i: (i, 0)),
        out_shape=jax.ShapeDtypeStruct((n_pad, _D), jnp.float32),
    )(eps_arr, parts, x, W1, b1.reshape(1, _D), W2, b2.reshape(1, _D))


def _mlp_final(parts, x2, eps, W1, b1, W2, b2, x0, h1, Wf, bf):
    n_pad = x2.shape[0]
    blk = 1024
    eps_arr = jnp.reshape(eps, (1, 1)).astype(jnp.float32)

    def body(eps_ref, p_ref, x2_ref, w1_ref, b1_ref, w2_ref, b2_ref,
             x0_ref, h1_ref, wf_ref, bf_ref, o_ref):
        a = p_ref[0] + p_ref[1] + (1.0 + eps_ref[0, 0]) * x2_ref[...]
        t = jnp.dot(a, w1_ref[...], preferred_element_type=jnp.float32) + b1_ref[...]
        t = jnp.maximum(t, 0.0)
        h3 = jnp.dot(t, w2_ref[...], preferred_element_type=jnp.float32) + b2_ref[...]
        acc = jnp.dot(x0_ref[...], wf_ref[0:_D], preferred_element_type=jnp.float32)
        acc += jnp.dot(h1_ref[...], wf_ref[_D:2 * _D], preferred_element_type=jnp.float32)
        acc += jnp.dot(x2_ref[...], wf_ref[2 * _D:3 * _D], preferred_element_type=jnp.float32)
        acc += jnp.dot(h3, wf_ref[3 * _D:4 * _D], preferred_element_type=jnp.float32)
        o_ref[...] = acc + bf_ref[...]

    return pl.pallas_call(
        body,
        grid=(n_pad // blk,),
        in_specs=[
            pl.BlockSpec(memory_space=pltpu.SMEM),
            pl.BlockSpec((_NC, blk, _D), lambda i: (0, i, 0)),
            pl.BlockSpec((blk, _D), lambda i: (i, 0)),
            pl.BlockSpec((_D, _D), lambda i: (0, 0)),
            pl.BlockSpec((1, _D), lambda i: (0, 0)),
            pl.BlockSpec((_D, _D), lambda i: (0, 0)),
            pl.BlockSpec((1, _D), lambda i: (0, 0)),
            pl.BlockSpec((blk, _D), lambda i: (i, 0)),
            pl.BlockSpec((blk, _D), lambda i: (i, 0)),
            pl.BlockSpec((4 * _D, _D), lambda i: (0, 0)),
            pl.BlockSpec((1, _D), lambda i: (0, 0)),
        ],
        out_specs=pl.BlockSpec((blk, _D), lambda i: (i, 0)),
        out_shape=jax.ShapeDtypeStruct((n_pad, _D), jnp.float32),
    )(eps_arr, parts, x2, W1, b1.reshape(1, _D), W2, b2.reshape(1, _D),
      x0, h1, Wf, bf.reshape(1, _D))


def kernel(x, edge_index, eps0, W1_0, b1_0, W2_0, b2_0, eps1, W1_1, b1_1,
           W2_1, b2_1, eps2, W1_2, b1_2, W2_2, b2_2, Wf, bf):
    n = x.shape[0]
    e = edge_index.shape[1]
    blk = 1024
    n_pad = -(-(n + 1) // blk) * blk
    egrain = _NS * _CHUNK
    e_pad = -(-e // egrain) * egrain

    src = edge_index[0].astype(jnp.int32)
    dst = edge_index[1].astype(jnp.int32)
    pad_idx = jnp.full((e_pad - e + _CHUNK,), n, dtype=jnp.int32)
    src_p = jnp.concatenate([src, pad_idx])
    dst_p = jnp.concatenate([dst, pad_idx])

    x_pad = jnp.concatenate([x, jnp.zeros((n_pad - n, _D), jnp.float32)])
    zeros_hbm = jnp.zeros((n_pad, _D), jnp.float32)

    parts1 = _sc_aggregate(x_pad, src_p, dst_p, zeros_hbm)
    h1 = _mlp(parts1, x_pad, eps0, W1_0, b1_0, W2_0, b2_0)
    parts2 = _sc_aggregate(h1, src_p, dst_p, zeros_hbm)
    h2 = _mlp(parts2, h1, eps1, W1_1, b1_1, W2_1, b2_1)
    parts3 = _sc_aggregate(h2, src_p, dst_p, zeros_hbm)
    out_pad = _mlp_final(parts3, h2, eps2, W1_2, b1_2, W2_2, b2_2,
                         x_pad, h1, Wf, bf)
    return out_pad[:n]


# speedup vs baseline: 1.3292x; 1.0410x over previous
import functools

import jax
import jax.numpy as jnp
from jax import lax
from jax.experimental import pallas as pl
from jax.experimental.pallas import tpu as pltpu
from jax.experimental.pallas import tpu_sc as plsc

_NC = 2
_NS = 16
_CHUNK = 128
_D = 128
_SPLIT0 = 0.56


def _sc_aggregate(x_hbm, src, dst, zeros_hbm):
    n_pad = x_hbm.shape[0]
    e_pad = src.shape[0] - _CHUNK
    tot = e_pad // (_NS * _CHUNK)
    cpt0 = int(tot * _SPLIT0 + 0.5)
    cpt1 = tot - cpt0
    rpt = n_pad // _NS

    mesh = plsc.VectorSubcoreMesh(core_axis_name="c", subcore_axis_name="s")

    @functools.partial(
        pl.kernel,
        out_type=jax.ShapeDtypeStruct((_NC, n_pad, _D), jnp.float32),
        mesh=mesh,
        scratch_types=[
            pltpu.VMEM((_CHUNK,), jnp.int32),
            pltpu.VMEM((_CHUNK,), jnp.int32),
            pltpu.VMEM((_CHUNK, _D), jnp.float32),
            pltpu.VMEM_SHARED((n_pad, _D), jnp.float32),
            pltpu.SemaphoreType.DMA,
            pltpu.SemaphoreType.DMA,
        ],
    )
    def agg_kernel(x_h, src_h, dst_h, z_h, out_h, sidx, didx, rows, acc, sem,
                   ssem):
        cid = lax.axis_index("c")
        sid = lax.axis_index("s")
        r0 = sid * rpt
        cpt = jnp.where(cid == 0, cpt0, cpt1)
        ebase = jnp.where(cid == 0, sid * cpt0,
                          _NS * cpt0 + sid * cpt1) * _CHUNK
        dummy = pl.multiple_of(e_pad, _CHUNK)
        pltpu.sync_copy(z_h.at[pl.ds(r0, rpt)], acc.at[pl.ds(r0, rpt)])
        plsc.subcore_barrier()
        pltpu.sync_copy(dst_h.at[pl.ds(dummy, _CHUNK)], didx)
        pltpu.async_copy(rows, acc.at[didx], ssem, add=True)

        def body(c, carry):
            off = pl.multiple_of(ebase + c * _CHUNK, _CHUNK)
            pltpu.sync_copy(src_h.at[pl.ds(off, _CHUNK)], sidx)
            pltpu.make_async_copy(rows, acc.at[didx], ssem).wait()
            g = pltpu.async_copy(x_h.at[sidx], rows, sem)
            pltpu.sync_copy(dst_h.at[pl.ds(off, _CHUNK)], didx)
            g.wait()
            pltpu.async_copy(rows, acc.at[didx], ssem, add=True)
            return carry

        lax.fori_loop(0, cpt, body, 0)
        pltpu.make_async_copy(rows, acc.at[didx], ssem).wait()
        plsc.subcore_barrier()
        pltpu.sync_copy(acc.at[pl.ds(r0, rpt)], out_h.at[cid, pl.ds(r0, rpt)])

    return agg_kernel(x_hbm, src, dst, zeros_hbm)


def _mlp(parts, x, eps, W1, b1, W2, b2):
    n_pad = x.shape[0]
    blk = 1024
    eps_arr = jnp.reshape(eps, (1, 1)).astype(jnp.float32)

    def body(eps_ref, p_ref, x_ref, w1_ref, b1_ref, w2_ref, b2_ref, o_ref):
        a = p_ref[0] + p_ref[1] + (1.0 + eps_ref[0, 0]) * x_ref[...]
        h = jnp.dot(a, w1_ref[...], preferred_element_type=jnp.float32) + b1_ref[...]
        h = jnp.maximum(h, 0.0)
        o_ref[...] = jnp.dot(h, w2_ref[...], preferred_element_type=jnp.float32) + b2_ref[...]

    return pl.pallas_call(
        body,
        grid=(n_pad // blk,),
        in_specs=[
            pl.BlockSpec(memory_space=pltpu.SMEM),
            pl.BlockSpec((_NC, blk, _D), lambda i: (0, i, 0)),
            pl.BlockSpec((blk, _D), lambda i: (i, 0)),
            pl.BlockSpec((_D, _D), lambda i: (0, 0)),
            pl.BlockSpec((1, _D), lambda i: (0, 0)),
            pl.BlockSpec((_D, _D), lambda i: (0, 0)),
            pl.BlockSpec((1, _D), lambda i: (0, 0)),
        ],
        out_specs=pl.BlockSpec((blk, _D), lambda i: (i, 0)),
        out_shape=jax.ShapeDtypeStruct((n_pad, _D), jnp.float32),
    )(eps_arr, parts, x, W1, b1.reshape(1, _D), W2, b2.reshape(1, _D))


def _mlp_final(parts, x2, eps, W1, b1, W2, b2, x0, h1, Wf, bf):
    n_pad = x2.shape[0]
    blk = 1024
    eps_arr = jnp.reshape(eps, (1, 1)).astype(jnp.float32)

    def body(eps_ref, p_ref, x2_ref, w1_ref, b1_ref, w2_ref, b2_ref,
             x0_ref, h1_ref, wf_ref, bf_ref, o_ref):
        a = p_ref[0] + p_ref[1] + (1.0 + eps_ref[0, 0]) * x2_ref[...]
        t = jnp.dot(a, w1_ref[...], preferred_element_type=jnp.float32) + b1_ref[...]
        t = jnp.maximum(t, 0.0)
        h3 = jnp.dot(t, w2_ref[...], preferred_element_type=jnp.float32) + b2_ref[...]
        acc = jnp.dot(x0_ref[...], wf_ref[0:_D], preferred_element_type=jnp.float32)
        acc += jnp.dot(h1_ref[...], wf_ref[_D:2 * _D], preferred_element_type=jnp.float32)
        acc += jnp.dot(x2_ref[...], wf_ref[2 * _D:3 * _D], preferred_element_type=jnp.float32)
        acc += jnp.dot(h3, wf_ref[3 * _D:4 * _D], preferred_element_type=jnp.float32)
        o_ref[...] = acc + bf_ref[...]

    return pl.pallas_call(
        body,
        grid=(n_pad // blk,),
        in_specs=[
            pl.BlockSpec(memory_space=pltpu.SMEM),
            pl.BlockSpec((_NC, blk, _D), lambda i: (0, i, 0)),
            pl.BlockSpec((blk, _D), lambda i: (i, 0)),
            pl.BlockSpec((_D, _D), lambda i: (0, 0)),
            pl.BlockSpec((1, _D), lambda i: (0, 0)),
            pl.BlockSpec((_D, _D), lambda i: (0, 0)),
            pl.BlockSpec((1, _D), lambda i: (0, 0)),
            pl.BlockSpec((blk, _D), lambda i: (i, 0)),
            pl.BlockSpec((blk, _D), lambda i: (i, 0)),
            pl.BlockSpec((4 * _D, _D), lambda i: (0, 0)),
            pl.BlockSpec((1, _D), lambda i: (0, 0)),
        ],
        out_specs=pl.BlockSpec((blk, _D), lambda i: (i, 0)),
        out_shape=jax.ShapeDtypeStruct((n_pad, _D), jnp.float32),
    )(eps_arr, parts, x2, W1, b1.reshape(1, _D), W2, b2.reshape(1, _D),
      x0, h1, Wf, bf.reshape(1, _D))


def kernel(x, edge_index, eps0, W1_0, b1_0, W2_0, b2_0, eps1, W1_1, b1_1,
           W2_1, b2_1, eps2, W1_2, b1_2, W2_2, b2_2, Wf, bf):
    n = x.shape[0]
    e = edge_index.shape[1]
    blk = 1024
    n_pad = -(-(n + 1) // blk) * blk
    egrain = _NS * _CHUNK
    e_pad = -(-e // egrain) * egrain

    src = edge_index[0].astype(jnp.int32)
    dst = edge_index[1].astype(jnp.int32)
    pad_idx = jnp.full((e_pad - e + _CHUNK,), n, dtype=jnp.int32)
    src_p = jnp.concatenate([src, pad_idx])
    dst_p = jnp.concatenate([dst, pad_idx])

    x_pad = jnp.concatenate([x, jnp.zeros((n_pad - n, _D), jnp.float32)])
    zeros_hbm = jnp.zeros((n_pad, _D), jnp.float32)

    parts1 = _sc_aggregate(x_pad, src_p, dst_p, zeros_hbm)
    h1 = _mlp(parts1, x_pad, eps0, W1_0, b1_0, W2_0, b2_0)
    parts2 = _sc_aggregate(h1, src_p, dst_p, zeros_hbm)
    h2 = _mlp(parts2, h1, eps1, W1_1, b1_1, W2_1, b2_1)
    parts3 = _sc_aggregate(h2, src_p, dst_p, zeros_hbm)
    out_pad = _mlp_final(parts3, h2, eps2, W1_2, b1_2, W2_2, b2_2,
                         x_pad, h1, Wf, bf)
    return out_pad[:n]


# speedup vs baseline: 1.3479x; 1.0141x over previous
import functools

import jax
import jax.numpy as jnp
from jax import lax
from jax.experimental import pallas as pl
from jax.experimental.pallas import tpu as pltpu
from jax.experimental.pallas import tpu_sc as plsc

_NC = 2
_NS = 16
_CHUNK = 128
_D = 128
_SPLIT0 = 0.60


def _sc_aggregate(x_hbm, src, dst, zeros_hbm):
    n_pad = x_hbm.shape[0]
    e_pad = src.shape[0] - _CHUNK
    tot = e_pad // (_NS * _CHUNK)
    cpt0 = int(tot * _SPLIT0 + 0.5)
    cpt1 = tot - cpt0
    rpt = n_pad // _NS

    mesh = plsc.VectorSubcoreMesh(core_axis_name="c", subcore_axis_name="s")

    @functools.partial(
        pl.kernel,
        out_type=jax.ShapeDtypeStruct((_NC, n_pad, _D), jnp.float32),
        mesh=mesh,
        scratch_types=[
            pltpu.VMEM((_CHUNK,), jnp.int32),
            pltpu.VMEM((_CHUNK,), jnp.int32),
            pltpu.VMEM((_CHUNK, _D), jnp.float32),
            pltpu.VMEM_SHARED((n_pad, _D), jnp.float32),
            pltpu.SemaphoreType.DMA,
            pltpu.SemaphoreType.DMA,
        ],
    )
    def agg_kernel(x_h, src_h, dst_h, z_h, out_h, sidx, didx, rows, acc, sem,
                   ssem):
        cid = lax.axis_index("c")
        sid = lax.axis_index("s")
        r0 = sid * rpt
        cpt = jnp.where(cid == 0, cpt0, cpt1)
        ebase = jnp.where(cid == 0, sid * cpt0,
                          _NS * cpt0 + sid * cpt1) * _CHUNK
        dummy = pl.multiple_of(e_pad, _CHUNK)
        pltpu.sync_copy(z_h.at[pl.ds(r0, rpt)], acc.at[pl.ds(r0, rpt)])
        plsc.subcore_barrier()
        pltpu.sync_copy(dst_h.at[pl.ds(dummy, _CHUNK)], didx)
        pltpu.async_copy(rows, acc.at[didx], ssem, add=True)

        def body(c, carry):
            off = pl.multiple_of(ebase + c * _CHUNK, _CHUNK)
            pltpu.sync_copy(src_h.at[pl.ds(off, _CHUNK)], sidx)
            pltpu.make_async_copy(rows, acc.at[didx], ssem).wait()
            g = pltpu.async_copy(x_h.at[sidx], rows, sem)
            pltpu.sync_copy(dst_h.at[pl.ds(off, _CHUNK)], didx)
            g.wait()
            pltpu.async_copy(rows, acc.at[didx], ssem, add=True)
            return carry

        lax.fori_loop(0, cpt, body, 0)
        pltpu.make_async_copy(rows, acc.at[didx], ssem).wait()
        plsc.subcore_barrier()
        pltpu.sync_copy(acc.at[pl.ds(r0, rpt)], out_h.at[cid, pl.ds(r0, rpt)])

    return agg_kernel(x_hbm, src, dst, zeros_hbm)


def _mlp(parts, x, eps, W1, b1, W2, b2):
    n_pad = x.shape[0]
    blk = 1024
    eps_arr = jnp.reshape(eps, (1, 1)).astype(jnp.float32)

    def body(eps_ref, p_ref, x_ref, w1_ref, b1_ref, w2_ref, b2_ref, o_ref):
        a = p_ref[0] + p_ref[1] + (1.0 + eps_ref[0, 0]) * x_ref[...]
        h = jnp.dot(a, w1_ref[...], preferred_element_type=jnp.float32) + b1_ref[...]
        h = jnp.maximum(h, 0.0)
        o_ref[...] = jnp.dot(h, w2_ref[...], preferred_element_type=jnp.float32) + b2_ref[...]

    return pl.pallas_call(
        body,
        grid=(n_pad // blk,),
        in_specs=[
            pl.BlockSpec(memory_space=pltpu.SMEM),
            pl.BlockSpec((_NC, blk, _D), lambda i: (0, i, 0)),
            pl.BlockSpec((blk, _D), lambda i: (i, 0)),
            pl.BlockSpec((_D, _D), lambda i: (0, 0)),
            pl.BlockSpec((1, _D), lambda i: (0, 0)),
            pl.BlockSpec((_D, _D), lambda i: (0, 0)),
            pl.BlockSpec((1, _D), lambda i: (0, 0)),
        ],
        out_specs=pl.BlockSpec((blk, _D), lambda i: (i, 0)),
        out_shape=jax.ShapeDtypeStruct((n_pad, _D), jnp.float32),
    )(eps_arr, parts, x, W1, b1.reshape(1, _D), W2, b2.reshape(1, _D))


def _mlp_final(parts, x2, eps, W1, b1, W2, b2, x0, h1, Wf, bf):
    n_pad = x2.shape[0]
    blk = 1024
    eps_arr = jnp.reshape(eps, (1, 1)).astype(jnp.float32)

    def body(eps_ref, p_ref, x2_ref, w1_ref, b1_ref, w2_ref, b2_ref,
             x0_ref, h1_ref, wf_ref, bf_ref, o_ref):
        a = p_ref[0] + p_ref[1] + (1.0 + eps_ref[0, 0]) * x2_ref[...]
        t = jnp.dot(a, w1_ref[...], preferred_element_type=jnp.float32) + b1_ref[...]
        t = jnp.maximum(t, 0.0)
        h3 = jnp.dot(t, w2_ref[...], preferred_element_type=jnp.float32) + b2_ref[...]
        acc = jnp.dot(x0_ref[...], wf_ref[0:_D], preferred_element_type=jnp.float32)
        acc += jnp.dot(h1_ref[...], wf_ref[_D:2 * _D], preferred_element_type=jnp.float32)
        acc += jnp.dot(x2_ref[...], wf_ref[2 * _D:3 * _D], preferred_element_type=jnp.float32)
        acc += jnp.dot(h3, wf_ref[3 * _D:4 * _D], preferred_element_type=jnp.float32)
        o_ref[...] = acc + bf_ref[...]

    return pl.pallas_call(
        body,
        grid=(n_pad // blk,),
        in_specs=[
            pl.BlockSpec(memory_space=pltpu.SMEM),
            pl.BlockSpec((_NC, blk, _D), lambda i: (0, i, 0)),
            pl.BlockSpec((blk, _D), lambda i: (i, 0)),
            pl.BlockSpec((_D, _D), lambda i: (0, 0)),
            pl.BlockSpec((1, _D), lambda i: (0, 0)),
            pl.BlockSpec((_D, _D), lambda i: (0, 0)),
            pl.BlockSpec((1, _D), lambda i: (0, 0)),
            pl.BlockSpec((blk, _D), lambda i: (i, 0)),
            pl.BlockSpec((blk, _D), lambda i: (i, 0)),
            pl.BlockSpec((4 * _D, _D), lambda i: (0, 0)),
            pl.BlockSpec((1, _D), lambda i: (0, 0)),
        ],
        out_specs=pl.BlockSpec((blk, _D), lambda i: (i, 0)),
        out_shape=jax.ShapeDtypeStruct((n_pad, _D), jnp.float32),
    )(eps_arr, parts, x2, W1, b1.reshape(1, _D), W2, b2.reshape(1, _D),
      x0, h1, Wf, bf.reshape(1, _D))


def kernel(x, edge_index, eps0, W1_0, b1_0, W2_0, b2_0, eps1, W1_1, b1_1,
           W2_1, b2_1, eps2, W1_2, b1_2, W2_2, b2_2, Wf, bf):
    n = x.shape[0]
    e = edge_index.shape[1]
    blk = 1024
    n_pad = -(-(n + 1) // blk) * blk
    egrain = _NS * _CHUNK
    e_pad = -(-e // egrain) * egrain

    src = edge_index[0].astype(jnp.int32)
    dst = edge_index[1].astype(jnp.int32)
    pad_idx = jnp.full((e_pad - e + _CHUNK,), n, dtype=jnp.int32)
    src_p = jnp.concatenate([src, pad_idx])
    dst_p = jnp.concatenate([dst, pad_idx])

    x_pad = jnp.concatenate([x, jnp.zeros((n_pad - n, _D), jnp.float32)])
    zeros_hbm = jnp.zeros((n_pad, _D), jnp.float32)

    parts1 = _sc_aggregate(x_pad, src_p, dst_p, zeros_hbm)
    h1 = _mlp(parts1, x_pad, eps0, W1_0, b1_0, W2_0, b2_0)
    parts2 = _sc_aggregate(h1, src_p, dst_p, zeros_hbm)
    h2 = _mlp(parts2, h1, eps1, W1_1, b1_1, W2_1, b2_1)
    parts3 = _sc_aggregate(h2, src_p, dst_p, zeros_hbm)
    out_pad = _mlp_final(parts3, h2, eps2, W1_2, b1_2, W2_2, b2_2,
                         x_pad, h1, Wf, bf)
    return out_pad[:n]
